# baseline (speedup 1.0000x reference)
import functools

import jax
import jax.numpy as jnp
from jax import lax
from jax.experimental import pallas as pl
from jax.experimental.pallas import tpu as pltpu
from jax.experimental.pallas import tpu_sc as plsc

N = 50000
E = 1600000
D = 33
DP = 40

NC = 2
NS = 16
NW = NC * NS
EW = E // NW
K = 40
NCHUNK = EW // K
SB = 10
SBK = SB * K
NSB = NCHUNK // SB
NB = 3
LG = 2
ZR = 3128


def _make_sc_scatter():
    mesh = plsc.VectorSubcoreMesh(core_axis_name="c", subcore_axis_name="s")

    @functools.partial(
        pl.kernel,
        out_type=jax.ShapeDtypeStruct((NC, N, DP), jnp.float32),
        mesh=mesh,
        compiler_params=pltpu.CompilerParams(use_tc_tiling_on_sc=False),
        scratch_types=[
            pltpu.VMEM((SBK,), jnp.int32),
            pltpu.VMEM((SB, K), jnp.int32),
            pltpu.VMEM((NB, K, DP), jnp.float32),
            pltpu.VMEM_SHARED((N, DP), jnp.float32),
            pltpu.SemaphoreType.DMA,
        ] + [pltpu.SemaphoreType.DMA] * (2 * NB),
    )
    def sc_scatter(state_hbm, src_hbm, dst2d_hbm, zeros_hbm, out_hbm,
                   idx_s, idx_d, rows, acc, semi, *sems):
        semg = sems[:NB]
        sems_ = sems[NB:]
        c = lax.axis_index("c")
        s = lax.axis_index("s")
        wid = s * NC + c

        r0 = jnp.minimum(s * ZR, N - ZR)
        pltpu.sync_copy(zeros_hbm, acc.at[pl.ds(r0, ZR)])
        plsc.subcore_barrier()

        base = wid * EW
        cbase = wid * NCHUNK

        def gather(sb, j):
            return pltpu.async_copy(
                state_hbm.at[idx_s.at[pl.ds(j * K, K)]],
                rows.at[j % NB], semg[j % NB])

        def scatter(j):
            return pltpu.async_copy(
                rows.at[j % NB], acc.at[idx_d.at[j]], sems_[j % NB], add=True)

        def body(sb, carry):
            hi1 = pltpu.async_copy(
                src_hbm.at[pl.ds(base + sb * SBK, SBK)], idx_s, semi)
            hi2 = pltpu.async_copy(
                dst2d_hbm.at[pl.ds(cbase + sb * SB, SB)], idx_d, semi)
            hi1.wait()
            hi2.wait()
            hg = [None] * SB
            hs = [None] * SB
            for j in range(SB):
                if j >= NB:
                    hs[j - NB].wait()
                hg[j] = gather(sb, j)
                if j >= LG:
                    hg[j - LG].wait()
                    hs[j - LG] = scatter(j - LG)
            for j in range(SB - LG, SB):
                hg[j].wait()
                hs[j] = scatter(j)
            for j in range(SB - NB, SB):
                hs[j].wait()
            return carry

        lax.fori_loop(0, NSB, body, 0)
        plsc.subcore_barrier()

        pltpu.sync_copy(acc.at[pl.ds(r0, ZR)], out_hbm.at[c, pl.ds(r0, ZR)])

    return sc_scatter


_sc_scatter = _make_sc_scatter()


def _tc_combine_body(acc_ref, state_ref, wrel_ref, wroot_ref, b_ref, out_ref):
    agg = acc_ref[0] + acc_ref[1]
    out = jnp.dot(agg, wrel_ref[...], preferred_element_type=jnp.float32)
    out += jnp.dot(state_ref[...], wroot_ref[...],
                   preferred_element_type=jnp.float32)
    out += b_ref[...]
    out_ref[...] = jnp.maximum(out, 0.0)


BN = 5000


def _tc_combine(pacc, state, w_rel, w_root, b):
    grid = (N // BN,)
    return pl.pallas_call(
        _tc_combine_body,
        grid=grid,
        in_specs=[
            pl.BlockSpec((NC, BN, DP), lambda i: (0, i, 0)),
            pl.BlockSpec((BN, D), lambda i: (i, 0)),
            pl.BlockSpec((DP, D), lambda i: (0, 0)),
            pl.BlockSpec((D, D), lambda i: (0, 0)),
            pl.BlockSpec((1, D), lambda i: (0, 0)),
        ],
        out_specs=pl.BlockSpec((BN, D), lambda i: (i, 0)),
        out_shape=jax.ShapeDtypeStruct((N, D), jnp.float32),
    )(pacc, state, w_rel, w_root, b)


def kernel(state, edge_index, W1_rel, b1_rel, W1_root, W2_rel, b2_rel, W2_root):
    del W1_rel, b1_rel, W1_root
    src = edge_index[0].astype(jnp.int32)
    dst = edge_index[1].astype(jnp.int32).reshape(E // K, K)
    state_p = jnp.pad(state, ((0, 0), (0, DP - D)))
    zeros = jnp.zeros((ZR, DP), jnp.float32)
    pacc = _sc_scatter(state_p, src, dst, zeros)
    w_rel = jnp.pad(W2_rel, ((0, DP - D), (0, 0)))
    return _tc_combine(pacc, state, w_rel, W2_root, b2_rel.reshape(1, D))


# speedup vs baseline: 1.0497x; 1.0497x over previous
import functools

import jax
import jax.numpy as jnp
from jax import lax
from jax.experimental import pallas as pl
from jax.experimental.pallas import tpu as pltpu
from jax.experimental.pallas import tpu_sc as plsc

N = 50000
E = 1600000
D = 33
DP = 40

NC = 2
NS = 16
NW = NC * NS
EW = E // NW
K = 40
NCHUNK = EW // K
SB = 10
SBK = SB * K
NSB = NCHUNK // SB
NB = 3
LG = 2
ZR = 3128


def _make_sc_scatter():
    mesh = plsc.VectorSubcoreMesh(core_axis_name="c", subcore_axis_name="s")

    @functools.partial(
        pl.kernel,
        out_type=jax.ShapeDtypeStruct((NC, N, DP), jnp.float32),
        mesh=mesh,
        compiler_params=pltpu.CompilerParams(use_tc_tiling_on_sc=False),
        scratch_types=[
            pltpu.VMEM((SB, K), jnp.int32),
            pltpu.VMEM((SB, K), jnp.int32),
            pltpu.VMEM((NB, K, DP), jnp.float32),
            pltpu.VMEM_SHARED((N, DP), jnp.float32),
            pltpu.SemaphoreType.DMA,
        ] + [pltpu.SemaphoreType.DMA] * (2 * NB),
    )
    def sc_scatter(state_hbm, ei_hbm, zeros_hbm, out_hbm,
                   idx_s, idx_d, rows, acc, semi, *sems):
        semg = sems[:NB]
        sems_ = sems[NB:]
        src2d_hbm = ei_hbm.at[0]
        dst2d_hbm = ei_hbm.at[1]
        c = lax.axis_index("c")
        s = lax.axis_index("s")
        wid = s * NC + c

        r0 = jnp.minimum(s * ZR, N - ZR)
        pltpu.sync_copy(zeros_hbm, acc.at[pl.ds(r0, ZR)])
        plsc.subcore_barrier()

        cbase = wid * NCHUNK

        def gather(sb, j):
            return pltpu.async_copy(
                state_hbm.at[idx_s.at[j]],
                rows.at[j % NB], semg[j % NB])

        def scatter(j):
            return pltpu.async_copy(
                rows.at[j % NB], acc.at[idx_d.at[j]], sems_[j % NB], add=True)

        def body(sb, carry):
            hi1 = pltpu.async_copy(
                src2d_hbm.at[pl.ds(cbase + sb * SB, SB)], idx_s, semi)
            hi2 = pltpu.async_copy(
                dst2d_hbm.at[pl.ds(cbase + sb * SB, SB)], idx_d, semi)
            hi1.wait()
            hi2.wait()
            hg = [None] * SB
            hs = [None] * SB
            for j in range(SB):
                if j >= NB:
                    hs[j - NB].wait()
                hg[j] = gather(sb, j)
                if j >= LG:
                    hg[j - LG].wait()
                    hs[j - LG] = scatter(j - LG)
            for j in range(SB - LG, SB):
                hg[j].wait()
                hs[j] = scatter(j)
            for j in range(SB - NB, SB):
                hs[j].wait()
            return carry

        lax.fori_loop(0, NSB, body, 0)
        plsc.subcore_barrier()

        pltpu.sync_copy(acc.at[pl.ds(r0, ZR)], out_hbm.at[c, pl.ds(r0, ZR)])

    return sc_scatter


_sc_scatter = _make_sc_scatter()


def _tc_combine_body(acc_ref, state_ref, wrel_ref, wroot_ref, b_ref, out_ref):
    agg = acc_ref[0] + acc_ref[1]
    out = jnp.dot(agg, wrel_ref[...], preferred_element_type=jnp.float32)
    out += jnp.dot(state_ref[...], wroot_ref[...],
                   preferred_element_type=jnp.float32)
    out += b_ref[...]
    out_ref[...] = jnp.maximum(out, 0.0)


BN = 5000


def _tc_combine(pacc, state, w_rel, w_root, b):
    grid = (N // BN,)
    return pl.pallas_call(
        _tc_combine_body,
        grid=grid,
        in_specs=[
            pl.BlockSpec((NC, BN, DP), lambda i: (0, i, 0)),
            pl.BlockSpec((BN, D), lambda i: (i, 0)),
            pl.BlockSpec((DP, D), lambda i: (0, 0)),
            pl.BlockSpec((D, D), lambda i: (0, 0)),
            pl.BlockSpec((1, D), lambda i: (0, 0)),
        ],
        out_specs=pl.BlockSpec((BN, D), lambda i: (i, 0)),
        out_shape=jax.ShapeDtypeStruct((N, D), jnp.float32),
    )(pacc, state, w_rel, w_root, b)


def kernel(state, edge_index, W1_rel, b1_rel, W1_root, W2_rel, b2_rel, W2_root):
    del W1_rel, b1_rel, W1_root
    ei3 = edge_index.astype(jnp.int32).reshape(2, E // K, K)
    state_p = jnp.pad(state, ((0, 0), (0, DP - D)))
    zeros = jnp.zeros((ZR, DP), jnp.float32)
    pacc = _sc_scatter(state_p, ei3, zeros)
    w_rel = jnp.pad(W2_rel, ((0, DP - D), (0, 0)))
    return _tc_combine(pacc, state, w_rel, W2_root, b2_rel.reshape(1, D))
